# baseline (speedup 1.0000x reference)
import functools

import jax
import jax.numpy as jnp
from jax import lax
from jax.experimental import pallas as pl
from jax.experimental.pallas import tpu as pltpu
from jax.experimental.pallas import tpu_sc as plsc

_CHUNK = 72
_LANES = 16
_NROW = 3
_UNROLL = 6
_ZBLK = 40
_GROUPS = [(0, 0), (16, 0), (32, 0), (48, 0), (56, 8)]


def _out_body(p_ref, w_ref, o_ref):
    h = p_ref[0] + p_ref[1]
    t = jnp.dot(h, w_ref[...], preferred_element_type=jnp.float32)
    o_ref[...] = jnp.where(t >= 0.0, t, 0.2 * t)


@functools.cache
def _sc_spmm(n_nodes, n_edges, feat, nc, ns):
    nw = nc * ns
    epw = n_edges // nw
    assert n_edges % nw == 0 and epw % _CHUNK == 0
    n_chunks = epw // _CHUNK
    n_main = (n_chunks // _UNROLL) * _UNROLL
    n_tail = n_chunks - n_main
    assert n_main >= 2 * _UNROLL
    assert n_nodes % _ZBLK == 0
    n_blocks = n_nodes // _ZBLK
    blocks_per_tile = -(-n_blocks // ns)
    n_vec = feat // _LANES
    n_grp = _CHUNK // _LANES

    mesh = plsc.VectorSubcoreMesh(core_axis_name="c", subcore_axis_name="s")

    @functools.partial(
        pl.kernel,
        mesh=mesh,
        out_type=jax.ShapeDtypeStruct((nc, n_nodes, feat), jnp.float32),
        scratch_types=(
            [
                pltpu.VMEM((epw,), jnp.int32),
                pltpu.VMEM((epw,), jnp.float32),
            ]
            + [pltpu.VMEM((_CHUNK, feat), jnp.float32)] * _NROW
            + [pltpu.VMEM((_CHUNK,), jnp.int32)] * 2
            + [pltpu.VMEM((_CHUNK,), jnp.int32)] * 2
            + [pltpu.SemaphoreType.DMA] * (1 + 2 + _NROW + 2)
            + [pltpu.VMEM_SHARED((n_nodes, feat), jnp.float32)]
        ),
    )
    def spmm(xfeat, srcs, dsts, ew, out, src_v, w_v, *scr):
        rows_v = scr[:_NROW]
        dst_v = scr[_NROW:_NROW + 2]
        sdst = scr[_NROW + 2:_NROW + 4]
        sem_pre = scr[_NROW + 4]
        sem_ix = scr[_NROW + 5:_NROW + 7]
        sem_ga = scr[_NROW + 7:_NROW + 7 + _NROW]
        sem_sc = scr[_NROW + 7 + _NROW:_NROW + 9 + _NROW]
        acc = scr[_NROW + 9 + _NROW]

        c = lax.axis_index("c")
        s = lax.axis_index("s")
        wid = s * nc + c
        base0 = wid * epw

        pltpu.async_copy(srcs.at[pl.ds(base0, epw)], src_v, sem_pre)
        pltpu.async_copy(ew.at[pl.ds(base0, epw)], w_v, sem_pre)

        def start_dst(k, b):
            pltpu.async_copy(dsts.at[pl.ds(base0 + k * _CHUNK, _CHUNK)],
                             dst_v[b], sem_ix[b])

        def wait_dst(b):
            pltpu.make_async_copy(dsts.at[pl.ds(base0, _CHUNK)],
                                  dst_v[b], sem_ix[b]).wait()

        start_dst(0, 0)

        def zero_rows(e, carry):
            for j in range(n_vec):
                rows_v[0][e, pl.ds(j * _LANES, _LANES)] = (
                    jnp.zeros((_LANES,), jnp.float32))
            return carry
        lax.fori_loop(0, _ZBLK, zero_rows, 0)

        for i in range(blocks_per_tile):
            blk = s + i * ns

            @pl.when(blk < n_blocks)
            def _():
                pltpu.sync_copy(rows_v[0].at[pl.ds(0, _ZBLK)],
                                acc.at[pl.ds(blk * _ZBLK, _ZBLK)])

        pltpu.make_async_copy(srcs.at[pl.ds(base0, epw)], src_v, sem_pre).wait()
        pltpu.make_async_copy(ew.at[pl.ds(base0, epw)], w_v, sem_pre).wait()
        plsc.subcore_barrier()

        def start_gather(k, b):
            pltpu.async_copy(
                xfeat.at[src_v.at[pl.ds(k * _CHUNK, _CHUNK)]],
                rows_v[b], sem_ga[b])

        def wait_gather(b):
            pltpu.make_async_copy(
                xfeat.at[src_v.at[pl.ds(0, _CHUNK)]], rows_v[b],
                sem_ga[b]).wait()

        def start_scatter(b, p):
            pltpu.async_copy(rows_v[b], acc.at[sdst[p]], sem_sc[p], add=True)

        def wait_scatter(b, p):
            pltpu.make_async_copy(rows_v[b], acc.at[sdst[p]], sem_sc[p]).wait()

        def guard(cond, fn):
            if isinstance(cond, bool):
                if cond:
                    fn()
            else:
                pl.when(cond)(fn)

        def body(k, b3, b2):
            guard(k >= 2, lambda: wait_scatter((b3 + 1) % _NROW, b2))
            guard(k + 1 < n_chunks, lambda: start_dst(k + 1, (b2 + 1) % 2))
            guard(k + 1 < n_chunks,
                  lambda: start_gather(k + 1, (b3 + 1) % _NROW))
            wait_gather(b3)

            koff = k * _CHUNK

            def scale_group(goff, g0):
                wv = w_v[pl.ds(koff + goff, _LANES)]
                for e2 in range(g0, _LANES):
                    e = goff + e2
                    w = wv[e2]
                    for j in range(n_vec):
                        sl = pl.ds(j * _LANES, _LANES)
                        rows_v[b3][e, sl] = rows_v[b3][e, sl] * w

            def scale(g, c2):
                scale_group(g * _LANES, 0)
                return c2
            lax.fori_loop(0, 4, scale, 0)
            scale_group(56, 8)

            wait_dst(b2)
            for goff, _ in _GROUPS:
                gsl = pl.ds(goff, _LANES)
                sdst[b2][gsl] = dst_v[b2][gsl]

            start_scatter(b3, b2)

        start_gather(0, 0)

        def outer_body(o, carry):
            k0 = o * _UNROLL
            for u in range(_UNROLL):
                body(k0 + u, u % _NROW, u % 2)
            return carry
        lax.fori_loop(0, n_main // _UNROLL, outer_body, 0)

        for t in range(n_tail):
            k = n_main + t
            body(k, k % _NROW, k % 2)

        wait_scatter((n_chunks - 2) % _NROW, (n_chunks - 2) % 2)
        wait_scatter((n_chunks - 1) % _NROW, (n_chunks - 1) % 2)

        plsc.subcore_barrier()

        for i in range(blocks_per_tile):
            blk = s + i * ns

            @pl.when(blk < n_blocks)
            def _():
                sl = pl.ds(blk * _ZBLK, _ZBLK)
                pltpu.sync_copy(acc.at[sl], out.at[c, sl])

    return spmm


def kernel(features, edge_index, edge_weight, weight):
    n, f_in = features.shape
    f_out = weight.shape[1]
    e = edge_weight.shape[0]

    info = plsc.get_sparse_core_info()
    nw = info.num_cores * info.num_subcores
    unit = nw * _CHUNK
    e_pad = -(-e // unit) * unit
    src = edge_index[0]
    dst = edge_index[1]
    ew = edge_weight
    if e_pad != e:
        pad_idx = (jnp.arange(e_pad - e, dtype=jnp.int32) % n).astype(jnp.int32)
        src = jnp.concatenate([src, pad_idx])
        dst = jnp.concatenate([dst, pad_idx])
        ew = jnp.concatenate([ew, jnp.zeros((e_pad - e,), jnp.float32)])

    partials = _sc_spmm(n, e_pad, f_in, info.num_cores, info.num_subcores)(
        features, src, dst, ew)

    bm = 1000
    out = pl.pallas_call(
        _out_body,
        grid=(n // bm,),
        in_specs=[
            pl.BlockSpec((2, bm, f_in), lambda i: (0, i, 0)),
            pl.BlockSpec((f_in, f_out), lambda i: (0, 0)),
        ],
        out_specs=pl.BlockSpec((bm, f_out), lambda i: (i, 0)),
        out_shape=jax.ShapeDtypeStruct((n, f_out), jnp.float32),
    )(partials, weight)
    return out
